# baseline (speedup 1.0000x reference)
import functools

import jax
import jax.numpy as jnp
from jax import lax
from jax.experimental import pallas as pl
from jax.experimental.pallas import tpu as pltpu
from jax.experimental.pallas import tpu_sc as plsc

from jax._src import core as _jax_core
from jax._src.lib.mlir import ir as _ir
from jax._src.lib.mlir.dialects import arith as _arith
from jax._src.lib.mlir.dialects import vector as _vector
from jax._src.pallas.mosaic import core as _tpu_core
from jax._src.pallas.mosaic import lowering as _tc_lowering
from jax.experimental.mosaic.dialects import tpu as _tpu_d

_lane_xor_p = _jax_core.Primitive("sc_lane_xor_perm")


@_lane_xor_p.def_abstract_eval
def _lane_xor_abstract_eval(x, *, k):
    del k
    return x


@_tc_lowering.register_lowering_rule(
    _lane_xor_p, kernel_types=[_tpu_core.CoreType.SC_VECTOR_SUBCORE]
)
def _lane_xor_lowering(ctx, x, *, k):
    del ctx
    i32 = _ir.IntegerType.get_signless(32)
    vec_ty = _ir.VectorType.get([16], i32)
    c = _arith.constant(i32, _ir.IntegerAttr.get(i32, k))
    cv = _vector.broadcast(vec_ty, c)
    idx = _arith.xori(cv, _tpu_d.iota(vec_ty, dimensions=[0]))
    return _tpu_d.dynamic_gather(x, idx, dimensions=[0])


def _lane_xor(x, k):
    return _lane_xor_p.bind(x, k=k)

N = 100000
D = 128
S = 512
RB = 128
NW = 32
BPW = 13
M = BPW * NW * RB
NT = N - M
BT = 1024
GT = (NT + BT - 1) // BT
WIDTH = D + 16
WT = D + 8

_mesh = plsc.VectorSubcoreMesh(
    core_axis_name="c", subcore_axis_name="s", num_cores=2, num_subcores=16
)


@functools.partial(
    pl.kernel,
    out_type=jax.ShapeDtypeStruct((2, S, WIDTH), jnp.float32),
    mesh=_mesh,
    scratch_types=[
        pltpu.VMEM((2, RB, D), jnp.float32),
        pltpu.VMEM((2, RB), jnp.int32),
        pltpu.VMEM((2, RB, WIDTH), jnp.float32),
        pltpu.VMEM((WIDTH,), jnp.float32),
        pltpu.VMEM_SHARED((S, WIDTH), jnp.float32),
        pltpu.SemaphoreType.DMA((2,)),
        pltpu.SemaphoreType.DMA((2,)),
        pltpu.SemaphoreType.DMA((2,)),
    ],
    compiler_params=pltpu.CompilerParams(use_tc_tiling_on_sc=False),
)
def _sc_partials(
    feats_hbm, ids_hbm, wb_hbm, out_hbm,
    fbuf, ibuf, sbuf, wbuf, acc, dfsem, disem, ssem,
):
    c = lax.axis_index("c")
    s = lax.axis_index("s")
    w = c * 16 + s

    for r in range(32):
        for j in range(WIDTH // 16):
            sbuf[0, r, pl.ds(16 * j, 16)] = jnp.zeros((16,), jnp.float32)
    pltpu.sync_copy(sbuf.at[0, pl.ds(0, 32)], acc.at[pl.ds(s * 32, 32)])
    plsc.subcore_barrier()

    pltpu.sync_copy(wb_hbm, wbuf)
    wv = [wbuf[pl.ds(16 * j, 16)] for j in range(8)]
    bv = wbuf[pl.ds(128, 16)]
    dmask = lax.iota(jnp.int32, 16) == 0

    def lane_allsum(a):
        for k in (1, 2, 4, 8):
            a = a + _lane_xor(a, k)
        return a

    nblk = BPW

    def issue_in(t, buf):
        row0 = (w + 32 * t) * RB
        pltpu.async_copy(
            feats_hbm.at[pl.ds(row0, RB)], fbuf.at[buf], dfsem.at[buf]
        )
        pltpu.async_copy(ids_hbm.at[pl.ds(row0, RB)], ibuf.at[buf], disem.at[buf])

    issue_in(0, 0)

    def blk_body(t, carry):
        buf = lax.rem(t, 2)
        nbuf = 1 - buf

        @pl.when(t >= 1)
        def _():
            pltpu.make_async_copy(
                sbuf.at[nbuf], acc.at[ibuf.at[nbuf]], ssem.at[nbuf]
            ).wait()

        @pl.when(t + 1 < nblk)
        def _():
            issue_in(t + 1, nbuf)

        pltpu.make_async_copy(
            feats_hbm.at[pl.ds(0, RB)], fbuf.at[buf], dfsem.at[buf]
        ).wait()
        pltpu.make_async_copy(
            ids_hbm.at[pl.ds(0, RB)], ibuf.at[buf], disem.at[buf]
        ).wait()

        def group_body(g, carry2):
            rows = []
            for k in range(8):
                r = g * 8 + k
                fs = [fbuf[buf, r, pl.ds(16 * j, 16)] for j in range(8)]
                p0 = fs[0] * wv[0] + fs[1] * wv[1]
                p1 = fs[2] * wv[2] + fs[3] * wv[3]
                p2 = fs[4] * wv[4] + fs[5] * wv[5]
                p3 = fs[6] * wv[6] + fs[7] * wv[7]
                a = (p0 + p1) + (p2 + p3)
                rows.append((r, fs, a))
            for r, fs, a in rows:
                sv = lane_allsum(a) + bv
                w16 = 1.0 / (1.0 + jnp.exp(-sv))
                for j in range(8):
                    sbuf[buf, r, pl.ds(16 * j, 16)] = fs[j] * w16
                sbuf[buf, r, pl.ds(128, 16)] = jnp.where(dmask, w16, 0.0)
            return carry2

        lax.fori_loop(0, RB // 8, group_body, 0)
        pltpu.async_copy(sbuf.at[buf], acc.at[ibuf.at[buf]], ssem.at[buf], add=True)
        return carry

    lax.fori_loop(0, nblk, blk_body, 0)
    lastbuf = lax.rem(nblk - 1, 2)
    pltpu.make_async_copy(
        sbuf.at[lastbuf], acc.at[ibuf.at[lastbuf]], ssem.at[lastbuf]
    ).wait()
    plsc.subcore_barrier()
    pltpu.sync_copy(acc.at[pl.ds(s * 32, 32)], out_hbm.at[c, pl.ds(s * 32, 32)])


def _tc_partial_body(feats_ref, ids_ref, w_ref, b_ref, out_ref):
    i = pl.program_id(0)
    f = feats_ref[...]
    logits = lax.dot_general(
        w_ref[...], f, (((1,), (1,)), ((), ())),
        preferred_element_type=jnp.float32,
    )
    wt = 1.0 / (1.0 + jnp.exp(-(logits + b_ref[0])))
    valid = (lax.broadcasted_iota(jnp.int32, (1, BT), 1) + i * BT) < NT
    wt = jnp.where(valid, wt, 0.0)
    wtc = wt.reshape(BT, 1)
    gid = lax.broadcasted_iota(jnp.int32, (S, BT), 0)
    onehot = jnp.where(gid == ids_ref[...], 1.0, 0.0)
    aug = jnp.concatenate(
        [f * wtc, jnp.broadcast_to(wtc, (BT, 8))], axis=1
    )
    nd = jnp.dot(onehot, aug, preferred_element_type=jnp.float32)

    @pl.when(i == 0)
    def _():
        out_ref[...] = jnp.zeros((S, WT), jnp.float32)

    out_ref[...] += nd


_tc_partial = pl.pallas_call(
    _tc_partial_body,
    grid=(GT,),
    in_specs=[
        pl.BlockSpec((BT, D), lambda i: (M // BT + i, 0)),
        pl.BlockSpec((1, BT), lambda i: (0, M // BT + i)),
        pl.BlockSpec((1, D), lambda i: (0, 0)),
        pl.BlockSpec(memory_space=pltpu.SMEM),
    ],
    out_specs=pl.BlockSpec((S, WT), lambda i: (0, 0)),
    out_shape=jax.ShapeDtypeStruct((S, WT), jnp.float32),
)


def _tc_combine_body(parts_ref, tcp_ref, out_ref):
    p = parts_ref[0] + parts_ref[1]
    t = tcp_ref[...]
    num = p[:, :D] + t[:, :D]
    den = p[:, D:D + 1] + t[:, D:D + 1]
    out_ref[...] = num / (den + 1e-12)


_tc_combine = pl.pallas_call(
    _tc_combine_body,
    in_specs=[
        pl.BlockSpec(memory_space=pltpu.MemorySpace.VMEM),
        pl.BlockSpec(memory_space=pltpu.MemorySpace.VMEM),
    ],
    out_shape=jax.ShapeDtypeStruct((S, D), jnp.float32),
)


@jax.jit
def kernel(feats_node, segment_ids, W, b):
    ids = segment_ids.astype(jnp.int32)
    wb = jnp.concatenate(
        [W.reshape(D), jnp.broadcast_to(b.reshape(1), (16,))]
    )
    parts = _sc_partials(feats_node, ids, wb)
    tcp = _tc_partial(
        feats_node, ids.reshape(1, N), W.reshape(1, D), b.reshape(1)
    )
    return _tc_combine(parts, tcp)


# speedup vs baseline: 1.0558x; 1.0558x over previous
import functools

import jax
import jax.numpy as jnp
from jax import lax
from jax.experimental import pallas as pl
from jax.experimental.pallas import tpu as pltpu
from jax.experimental.pallas import tpu_sc as plsc

from jax._src import core as _jax_core
from jax._src.lib.mlir import ir as _ir
from jax._src.lib.mlir.dialects import arith as _arith
from jax._src.lib.mlir.dialects import vector as _vector
from jax._src.pallas.mosaic import core as _tpu_core
from jax._src.pallas.mosaic import lowering as _tc_lowering
from jax.experimental.mosaic.dialects import tpu as _tpu_d

_lane_xor_p = _jax_core.Primitive("sc_lane_xor_perm")


@_lane_xor_p.def_abstract_eval
def _lane_xor_abstract_eval(x, *, k):
    del k
    return x


@_tc_lowering.register_lowering_rule(
    _lane_xor_p, kernel_types=[_tpu_core.CoreType.SC_VECTOR_SUBCORE]
)
def _lane_xor_lowering(ctx, x, *, k):
    del ctx
    i32 = _ir.IntegerType.get_signless(32)
    vec_ty = _ir.VectorType.get([16], i32)
    c = _arith.constant(i32, _ir.IntegerAttr.get(i32, k))
    cv = _vector.broadcast(vec_ty, c)
    idx = _arith.xori(cv, _tpu_d.iota(vec_ty, dimensions=[0]))
    return _tpu_d.dynamic_gather(x, idx, dimensions=[0])


def _lane_xor(x, k):
    return _lane_xor_p.bind(x, k=k)

N = 100000
D = 128
S = 512
RB = 128
NW = 32
BPW = 14
M = BPW * NW * RB
NT = N - M
BT = 1024
GT = (NT + BT - 1) // BT
WIDTH = D + 16
WT = D + 8

_mesh = plsc.VectorSubcoreMesh(
    core_axis_name="c", subcore_axis_name="s", num_cores=2, num_subcores=16
)


@functools.partial(
    pl.kernel,
    out_type=jax.ShapeDtypeStruct((2, S, WIDTH), jnp.float32),
    mesh=_mesh,
    scratch_types=[
        pltpu.VMEM((2, RB, D), jnp.float32),
        pltpu.VMEM((2, RB), jnp.int32),
        pltpu.VMEM((2, RB, WIDTH), jnp.float32),
        pltpu.VMEM((WIDTH,), jnp.float32),
        pltpu.VMEM_SHARED((S, WIDTH), jnp.float32),
        pltpu.SemaphoreType.DMA((2,)),
        pltpu.SemaphoreType.DMA((2,)),
        pltpu.SemaphoreType.DMA((2,)),
    ],
    compiler_params=pltpu.CompilerParams(use_tc_tiling_on_sc=False),
)
def _sc_partials(
    feats_hbm, ids_hbm, wb_hbm, out_hbm,
    fbuf, ibuf, sbuf, wbuf, acc, dfsem, disem, ssem,
):
    c = lax.axis_index("c")
    s = lax.axis_index("s")
    w = c * 16 + s

    for r in range(32):
        for j in range(WIDTH // 16):
            sbuf[0, r, pl.ds(16 * j, 16)] = jnp.zeros((16,), jnp.float32)
    pltpu.sync_copy(sbuf.at[0, pl.ds(0, 32)], acc.at[pl.ds(s * 32, 32)])
    plsc.subcore_barrier()

    pltpu.sync_copy(wb_hbm, wbuf)
    wv = [wbuf[pl.ds(16 * j, 16)] for j in range(8)]
    bv = wbuf[pl.ds(128, 16)]
    dmask = lax.iota(jnp.int32, 16) == 0

    def lane_allsum(a):
        for k in (1, 2, 4, 8):
            a = a + _lane_xor(a, k)
        return a

    nblk = BPW

    def issue_in(t, buf):
        row0 = (w + 32 * t) * RB
        pltpu.async_copy(
            feats_hbm.at[pl.ds(row0, RB)], fbuf.at[buf], dfsem.at[buf]
        )
        pltpu.async_copy(ids_hbm.at[pl.ds(row0, RB)], ibuf.at[buf], disem.at[buf])

    issue_in(0, 0)

    def blk_body(t, carry):
        buf = lax.rem(t, 2)
        nbuf = 1 - buf

        @pl.when(t >= 1)
        def _():
            pltpu.make_async_copy(
                sbuf.at[nbuf], acc.at[ibuf.at[nbuf]], ssem.at[nbuf]
            ).wait()

        @pl.when(t + 1 < nblk)
        def _():
            issue_in(t + 1, nbuf)

        pltpu.make_async_copy(
            feats_hbm.at[pl.ds(0, RB)], fbuf.at[buf], dfsem.at[buf]
        ).wait()
        pltpu.make_async_copy(
            ids_hbm.at[pl.ds(0, RB)], ibuf.at[buf], disem.at[buf]
        ).wait()

        def group_body(g, carry2):
            rows = []
            for k in range(8):
                r = g * 8 + k
                fs = [fbuf[buf, r, pl.ds(16 * j, 16)] for j in range(8)]
                p0 = fs[0] * wv[0]
                p1 = fs[1] * wv[1]
                for j in range(2, 8, 2):
                    p0 = p0 + fs[j] * wv[j]
                    p1 = p1 + fs[j + 1] * wv[j + 1]
                a = p0 + p1
                rows.append((r, fs, a))
            for r, fs, a in rows:
                sv = lane_allsum(a) + bv
                w16 = 1.0 / (1.0 + jnp.exp(-sv))
                for j in range(8):
                    sbuf[buf, r, pl.ds(16 * j, 16)] = fs[j] * w16
                sbuf[buf, r, pl.ds(128, 16)] = jnp.where(dmask, w16, 0.0)
            return carry2

        lax.fori_loop(0, RB // 8, group_body, 0)
        pltpu.async_copy(sbuf.at[buf], acc.at[ibuf.at[buf]], ssem.at[buf], add=True)
        return carry

    lax.fori_loop(0, nblk, blk_body, 0)
    lastbuf = lax.rem(nblk - 1, 2)
    pltpu.make_async_copy(
        sbuf.at[lastbuf], acc.at[ibuf.at[lastbuf]], ssem.at[lastbuf]
    ).wait()
    plsc.subcore_barrier()
    pltpu.sync_copy(acc.at[pl.ds(s * 32, 32)], out_hbm.at[c, pl.ds(s * 32, 32)])


def _tc_partial_body(feats_ref, ids_ref, w_ref, b_ref, out_ref):
    i = pl.program_id(0)
    f = feats_ref[...]
    logits = lax.dot_general(
        w_ref[...], f, (((1,), (1,)), ((), ())),
        preferred_element_type=jnp.float32,
    )
    wt = 1.0 / (1.0 + jnp.exp(-(logits + b_ref[0])))
    valid = (lax.broadcasted_iota(jnp.int32, (1, BT), 1) + i * BT) < NT
    wt = jnp.where(valid, wt, 0.0)
    wtc = wt.reshape(BT, 1)
    gid = lax.broadcasted_iota(jnp.int32, (S, BT), 0)
    onehot = jnp.where(gid == ids_ref[...], 1.0, 0.0)
    aug = jnp.concatenate(
        [f * wtc, jnp.broadcast_to(wtc, (BT, 8))], axis=1
    )
    nd = jnp.dot(onehot, aug, preferred_element_type=jnp.float32)

    @pl.when(i == 0)
    def _():
        out_ref[...] = jnp.zeros((S, WT), jnp.float32)

    out_ref[...] += nd


_tc_partial = pl.pallas_call(
    _tc_partial_body,
    grid=(GT,),
    in_specs=[
        pl.BlockSpec((BT, D), lambda i: (M // BT + i, 0)),
        pl.BlockSpec((1, BT), lambda i: (0, M // BT + i)),
        pl.BlockSpec((1, D), lambda i: (0, 0)),
        pl.BlockSpec(memory_space=pltpu.SMEM),
    ],
    out_specs=pl.BlockSpec((S, WT), lambda i: (0, 0)),
    out_shape=jax.ShapeDtypeStruct((S, WT), jnp.float32),
)


def _tc_combine_body(parts_ref, tcp_ref, out_ref):
    p = parts_ref[0] + parts_ref[1]
    t = tcp_ref[...]
    num = p[:, :D] + t[:, :D]
    den = p[:, D:D + 1] + t[:, D:D + 1]
    out_ref[...] = num / (den + 1e-12)


_tc_combine = pl.pallas_call(
    _tc_combine_body,
    in_specs=[
        pl.BlockSpec(memory_space=pltpu.MemorySpace.VMEM),
        pl.BlockSpec(memory_space=pltpu.MemorySpace.VMEM),
    ],
    out_shape=jax.ShapeDtypeStruct((S, D), jnp.float32),
)


@jax.jit
def kernel(feats_node, segment_ids, W, b):
    ids = segment_ids.astype(jnp.int32)
    wb = jnp.concatenate(
        [W.reshape(D), jnp.broadcast_to(b.reshape(1), (16,))]
    )
    parts = _sc_partials(feats_node, ids, wb)
    tcp = _tc_partial(
        feats_node, ids.reshape(1, N), W.reshape(1, D), b.reshape(1)
    )
    return _tc_combine(parts, tcp)


# speedup vs baseline: 1.0925x; 1.0348x over previous
import functools

import jax
import jax.numpy as jnp
from jax import lax
from jax.experimental import pallas as pl
from jax.experimental.pallas import tpu as pltpu
from jax.experimental.pallas import tpu_sc as plsc

from jax._src import core as _jax_core
from jax._src.lib.mlir import ir as _ir
from jax._src.lib.mlir.dialects import arith as _arith
from jax._src.lib.mlir.dialects import vector as _vector
from jax._src.pallas.mosaic import core as _tpu_core
from jax._src.pallas.mosaic import lowering as _tc_lowering
from jax.experimental.mosaic.dialects import tpu as _tpu_d

_lane_xor_p = _jax_core.Primitive("sc_lane_xor_perm")


@_lane_xor_p.def_abstract_eval
def _lane_xor_abstract_eval(x, *, k):
    del k
    return x


@_tc_lowering.register_lowering_rule(
    _lane_xor_p, kernel_types=[_tpu_core.CoreType.SC_VECTOR_SUBCORE]
)
def _lane_xor_lowering(ctx, x, *, k):
    del ctx
    i32 = _ir.IntegerType.get_signless(32)
    vec_ty = _ir.VectorType.get([16], i32)
    c = _arith.constant(i32, _ir.IntegerAttr.get(i32, k))
    cv = _vector.broadcast(vec_ty, c)
    idx = _arith.xori(cv, _tpu_d.iota(vec_ty, dimensions=[0]))
    return _tpu_d.dynamic_gather(x, idx, dimensions=[0])


def _lane_xor(x, k):
    return _lane_xor_p.bind(x, k=k)

N = 100000
D = 128
S = 512
RB = 128
NW = 32
BPW = 14
M = BPW * NW * RB
NT = N - M
BT = 2048
GT = (NT + BT - 1) // BT
WIDTH = D + 16
WT = D + 8

_mesh = plsc.VectorSubcoreMesh(
    core_axis_name="c", subcore_axis_name="s", num_cores=2, num_subcores=16
)


@functools.partial(
    pl.kernel,
    out_type=jax.ShapeDtypeStruct((2, S, WIDTH), jnp.float32),
    mesh=_mesh,
    scratch_types=[
        pltpu.VMEM((2, RB, D), jnp.float32),
        pltpu.VMEM((2, RB), jnp.int32),
        pltpu.VMEM((2, RB, WIDTH), jnp.float32),
        pltpu.VMEM((WIDTH,), jnp.float32),
        pltpu.VMEM_SHARED((S, WIDTH), jnp.float32),
        pltpu.SemaphoreType.DMA((2,)),
        pltpu.SemaphoreType.DMA((2,)),
        pltpu.SemaphoreType.DMA((2,)),
    ],
    compiler_params=pltpu.CompilerParams(use_tc_tiling_on_sc=False),
)
def _sc_partials(
    feats_hbm, ids_hbm, wb_hbm, out_hbm,
    fbuf, ibuf, sbuf, wbuf, acc, dfsem, disem, ssem,
):
    c = lax.axis_index("c")
    s = lax.axis_index("s")
    w = c * 16 + s

    for r in range(32):
        for j in range(WIDTH // 16):
            sbuf[0, r, pl.ds(16 * j, 16)] = jnp.zeros((16,), jnp.float32)
    pltpu.sync_copy(sbuf.at[0, pl.ds(0, 32)], acc.at[pl.ds(s * 32, 32)])
    plsc.subcore_barrier()

    pltpu.sync_copy(wb_hbm, wbuf)
    wv = [wbuf[pl.ds(16 * j, 16)] for j in range(8)]
    bv = wbuf[pl.ds(128, 16)]
    dmask = lax.iota(jnp.int32, 16) == 0

    def lane_allsum(a):
        for k in (1, 2, 4, 8):
            a = a + _lane_xor(a, k)
        return a

    nblk = BPW

    def issue_in(t, buf):
        row0 = (w + 32 * t) * RB
        pltpu.async_copy(
            feats_hbm.at[pl.ds(row0, RB)], fbuf.at[buf], dfsem.at[buf]
        )
        pltpu.async_copy(ids_hbm.at[pl.ds(row0, RB)], ibuf.at[buf], disem.at[buf])

    issue_in(0, 0)

    def blk_body(t, carry):
        buf = lax.rem(t, 2)
        nbuf = 1 - buf

        @pl.when(t >= 1)
        def _():
            pltpu.make_async_copy(
                sbuf.at[nbuf], acc.at[ibuf.at[nbuf]], ssem.at[nbuf]
            ).wait()

        @pl.when(t + 1 < nblk)
        def _():
            issue_in(t + 1, nbuf)

        pltpu.make_async_copy(
            feats_hbm.at[pl.ds(0, RB)], fbuf.at[buf], dfsem.at[buf]
        ).wait()
        pltpu.make_async_copy(
            ids_hbm.at[pl.ds(0, RB)], ibuf.at[buf], disem.at[buf]
        ).wait()

        def group_body(g, carry2):
            rows = []
            for k in range(8):
                r = g * 8 + k
                fs = [fbuf[buf, r, pl.ds(16 * j, 16)] for j in range(8)]
                p0 = fs[0] * wv[0]
                p1 = fs[1] * wv[1]
                for j in range(2, 8, 2):
                    p0 = p0 + fs[j] * wv[j]
                    p1 = p1 + fs[j + 1] * wv[j + 1]
                a = p0 + p1
                rows.append((r, fs, a))
            for r, fs, a in rows:
                sv = lane_allsum(a) + bv
                w16 = 1.0 / (1.0 + jnp.exp(-sv))
                for j in range(8):
                    sbuf[buf, r, pl.ds(16 * j, 16)] = fs[j] * w16
                sbuf[buf, r, pl.ds(128, 16)] = jnp.where(dmask, w16, 0.0)
            return carry2

        lax.fori_loop(0, RB // 8, group_body, 0)
        pltpu.async_copy(sbuf.at[buf], acc.at[ibuf.at[buf]], ssem.at[buf], add=True)
        return carry

    lax.fori_loop(0, nblk, blk_body, 0)
    lastbuf = lax.rem(nblk - 1, 2)
    pltpu.make_async_copy(
        sbuf.at[lastbuf], acc.at[ibuf.at[lastbuf]], ssem.at[lastbuf]
    ).wait()
    plsc.subcore_barrier()
    pltpu.sync_copy(acc.at[pl.ds(s * 32, 32)], out_hbm.at[c, pl.ds(s * 32, 32)])


def _tc_partial_body(feats_ref, ids_ref, w_ref, b_ref, out_ref):
    i = pl.program_id(0)
    f = feats_ref[...]
    logits = lax.dot_general(
        w_ref[...], f, (((1,), (1,)), ((), ())),
        preferred_element_type=jnp.float32,
    )
    wt = 1.0 / (1.0 + jnp.exp(-(logits + b_ref[0])))
    valid = (lax.broadcasted_iota(jnp.int32, (1, BT), 1) + i * BT) < NT
    wt = jnp.where(valid, wt, 0.0)
    wtc = wt.reshape(BT, 1)
    gid = lax.broadcasted_iota(jnp.int32, (S, BT), 0)
    onehot = jnp.where(gid == ids_ref[...], 1.0, 0.0)
    aug = jnp.concatenate(
        [f * wtc, jnp.broadcast_to(wtc, (BT, 8))], axis=1
    )
    nd = jnp.dot(onehot, aug, preferred_element_type=jnp.float32)

    @pl.when(i == 0)
    def _():
        out_ref[...] = jnp.zeros((S, WT), jnp.float32)

    out_ref[...] += nd


_tc_partial = pl.pallas_call(
    _tc_partial_body,
    grid=(GT,),
    in_specs=[
        pl.BlockSpec((BT, D), lambda i: (M // BT + i, 0)),
        pl.BlockSpec((1, BT), lambda i: (0, M // BT + i)),
        pl.BlockSpec((1, D), lambda i: (0, 0)),
        pl.BlockSpec(memory_space=pltpu.SMEM),
    ],
    out_specs=pl.BlockSpec((S, WT), lambda i: (0, 0)),
    out_shape=jax.ShapeDtypeStruct((S, WT), jnp.float32),
)


def _tc_combine_body(parts_ref, tcp_ref, out_ref):
    p = parts_ref[0] + parts_ref[1]
    t = tcp_ref[...]
    num = p[:, :D] + t[:, :D]
    den = p[:, D:D + 1] + t[:, D:D + 1]
    out_ref[...] = num / (den + 1e-12)


_tc_combine = pl.pallas_call(
    _tc_combine_body,
    in_specs=[
        pl.BlockSpec(memory_space=pltpu.MemorySpace.VMEM),
        pl.BlockSpec(memory_space=pltpu.MemorySpace.VMEM),
    ],
    out_shape=jax.ShapeDtypeStruct((S, D), jnp.float32),
)


@jax.jit
def kernel(feats_node, segment_ids, W, b):
    ids = segment_ids.astype(jnp.int32)
    wb = jnp.concatenate(
        [W.reshape(D), jnp.broadcast_to(b.reshape(1), (16,))]
    )
    parts = _sc_partials(feats_node, ids, wb)
    tcp = _tc_partial(
        feats_node, ids.reshape(1, N), W.reshape(1, D), b.reshape(1)
    )
    return _tc_combine(parts, tcp)


# speedup vs baseline: 1.1380x; 1.0416x over previous
import functools

import jax
import jax.numpy as jnp
from jax import lax
from jax.experimental import pallas as pl
from jax.experimental.pallas import tpu as pltpu
from jax.experimental.pallas import tpu_sc as plsc

from jax._src import core as _jax_core
from jax._src.lib.mlir import ir as _ir
from jax._src.lib.mlir.dialects import arith as _arith
from jax._src.lib.mlir.dialects import vector as _vector
from jax._src.pallas.mosaic import core as _tpu_core
from jax._src.pallas.mosaic import lowering as _tc_lowering
from jax.experimental.mosaic.dialects import tpu as _tpu_d

_lane_xor_p = _jax_core.Primitive("sc_lane_xor_perm")


@_lane_xor_p.def_abstract_eval
def _lane_xor_abstract_eval(x, *, k):
    del k
    return x


@_tc_lowering.register_lowering_rule(
    _lane_xor_p, kernel_types=[_tpu_core.CoreType.SC_VECTOR_SUBCORE]
)
def _lane_xor_lowering(ctx, x, *, k):
    del ctx
    i32 = _ir.IntegerType.get_signless(32)
    vec_ty = _ir.VectorType.get([16], i32)
    c = _arith.constant(i32, _ir.IntegerAttr.get(i32, k))
    cv = _vector.broadcast(vec_ty, c)
    idx = _arith.xori(cv, _tpu_d.iota(vec_ty, dimensions=[0]))
    return _tpu_d.dynamic_gather(x, idx, dimensions=[0])


def _lane_xor(x, k):
    return _lane_xor_p.bind(x, k=k)

N = 100000
D = 128
S = 512
RB = 128
NW = 32
BPW = 13
M = BPW * NW * RB
NT = N - M
BT = 2048
GT = (NT + BT - 1) // BT
WIDTH = D + 16
WT = D + 8

_mesh = plsc.VectorSubcoreMesh(
    core_axis_name="c", subcore_axis_name="s", num_cores=2, num_subcores=16
)


@functools.partial(
    pl.kernel,
    out_type=jax.ShapeDtypeStruct((2, S, WIDTH), jnp.float32),
    mesh=_mesh,
    scratch_types=[
        pltpu.VMEM((2, RB, D), jnp.float32),
        pltpu.VMEM((2, RB), jnp.int32),
        pltpu.VMEM((2, RB, WIDTH), jnp.float32),
        pltpu.VMEM((WIDTH,), jnp.float32),
        pltpu.VMEM_SHARED((S, WIDTH), jnp.float32),
        pltpu.SemaphoreType.DMA((2,)),
        pltpu.SemaphoreType.DMA((2,)),
        pltpu.SemaphoreType.DMA((2,)),
    ],
    compiler_params=pltpu.CompilerParams(use_tc_tiling_on_sc=False),
)
def _sc_partials(
    feats_hbm, ids_hbm, wb_hbm, out_hbm,
    fbuf, ibuf, sbuf, wbuf, acc, dfsem, disem, ssem,
):
    c = lax.axis_index("c")
    s = lax.axis_index("s")
    w = c * 16 + s

    for r in range(32):
        for j in range(WIDTH // 16):
            sbuf[0, r, pl.ds(16 * j, 16)] = jnp.zeros((16,), jnp.float32)
    pltpu.sync_copy(sbuf.at[0, pl.ds(0, 32)], acc.at[pl.ds(s * 32, 32)])
    plsc.subcore_barrier()

    pltpu.sync_copy(wb_hbm, wbuf)
    wv = [wbuf[pl.ds(16 * j, 16)] for j in range(8)]
    bv = wbuf[pl.ds(128, 16)]
    dmask = lax.iota(jnp.int32, 16) == 0

    def lane_allsum(a):
        for k in (1, 2, 4, 8):
            a = a + _lane_xor(a, k)
        return a

    nblk = BPW

    def issue_in(t, buf):
        row0 = (w + 32 * t) * RB
        pltpu.async_copy(
            feats_hbm.at[pl.ds(row0, RB)], fbuf.at[buf], dfsem.at[buf]
        )
        pltpu.async_copy(ids_hbm.at[pl.ds(row0, RB)], ibuf.at[buf], disem.at[buf])

    issue_in(0, 0)

    def blk_body(t, carry):
        buf = lax.rem(t, 2)
        nbuf = 1 - buf

        @pl.when(t >= 1)
        def _():
            pltpu.make_async_copy(
                sbuf.at[nbuf], acc.at[ibuf.at[nbuf]], ssem.at[nbuf]
            ).wait()

        @pl.when(t + 1 < nblk)
        def _():
            issue_in(t + 1, nbuf)

        pltpu.make_async_copy(
            feats_hbm.at[pl.ds(0, RB)], fbuf.at[buf], dfsem.at[buf]
        ).wait()
        pltpu.make_async_copy(
            ids_hbm.at[pl.ds(0, RB)], ibuf.at[buf], disem.at[buf]
        ).wait()

        def group_body(g, carry2):
            rows = []
            for k in range(8):
                r = g * 8 + k
                fs = [fbuf[buf, r, pl.ds(16 * j, 16)] for j in range(8)]
                p0 = fs[0] * wv[0]
                p1 = fs[1] * wv[1]
                for j in range(2, 8, 2):
                    p0 = p0 + fs[j] * wv[j]
                    p1 = p1 + fs[j + 1] * wv[j + 1]
                a = p0 + p1
                rows.append((r, fs, a))
            for r, fs, a in rows:
                sv = lane_allsum(a) + bv
                w16 = 1.0 / (1.0 + jnp.exp(-sv))
                for j in range(8):
                    sbuf[buf, r, pl.ds(16 * j, 16)] = fs[j] * w16
                sbuf[buf, r, pl.ds(128, 16)] = jnp.where(dmask, w16, 0.0)
            return carry2

        lax.fori_loop(0, RB // 8, group_body, 0)
        pltpu.async_copy(sbuf.at[buf], acc.at[ibuf.at[buf]], ssem.at[buf], add=True)
        return carry

    lax.fori_loop(0, nblk, blk_body, 0)
    lastbuf = lax.rem(nblk - 1, 2)
    pltpu.make_async_copy(
        sbuf.at[lastbuf], acc.at[ibuf.at[lastbuf]], ssem.at[lastbuf]
    ).wait()
    plsc.subcore_barrier()
    pltpu.sync_copy(acc.at[pl.ds(s * 32, 32)], out_hbm.at[c, pl.ds(s * 32, 32)])


def _tc_partial_body(feats_ref, ids_ref, w_ref, b_ref, out_ref):
    i = pl.program_id(0)
    f = feats_ref[...]
    logits = lax.dot_general(
        w_ref[...], f, (((1,), (1,)), ((), ())),
        preferred_element_type=jnp.float32,
    )
    wt = 1.0 / (1.0 + jnp.exp(-(logits + b_ref[0])))
    valid = (lax.broadcasted_iota(jnp.int32, (1, BT), 1) + i * BT) < NT
    wt = jnp.where(valid, wt, 0.0)
    wtc = wt.reshape(BT, 1)
    gid = lax.broadcasted_iota(jnp.int32, (S, BT), 0)
    onehot = jnp.where(gid == ids_ref[...], 1.0, 0.0)
    aug = jnp.concatenate(
        [f * wtc, jnp.broadcast_to(wtc, (BT, 8))], axis=1
    )
    nd = jnp.dot(onehot, aug, preferred_element_type=jnp.float32)

    @pl.when(i == 0)
    def _():
        out_ref[...] = jnp.zeros((S, WT), jnp.float32)

    out_ref[...] += nd


_tc_partial = pl.pallas_call(
    _tc_partial_body,
    grid=(GT,),
    in_specs=[
        pl.BlockSpec((BT, D), lambda i: (M // BT + i, 0)),
        pl.BlockSpec((1, BT), lambda i: (0, M // BT + i)),
        pl.BlockSpec((1, D), lambda i: (0, 0)),
        pl.BlockSpec(memory_space=pltpu.SMEM),
    ],
    out_specs=pl.BlockSpec((S, WT), lambda i: (0, 0)),
    out_shape=jax.ShapeDtypeStruct((S, WT), jnp.float32),
)


def _tc_combine_body(parts_ref, tcp_ref, out_ref):
    p = parts_ref[0] + parts_ref[1]
    t = tcp_ref[...]
    num = p[:, :D] + t[:, :D]
    den = p[:, D:D + 1] + t[:, D:D + 1]
    out_ref[...] = num / (den + 1e-12)


_tc_combine = pl.pallas_call(
    _tc_combine_body,
    in_specs=[
        pl.BlockSpec(memory_space=pltpu.MemorySpace.VMEM),
        pl.BlockSpec(memory_space=pltpu.MemorySpace.VMEM),
    ],
    out_shape=jax.ShapeDtypeStruct((S, D), jnp.float32),
)


@jax.jit
def kernel(feats_node, segment_ids, W, b):
    ids = segment_ids.astype(jnp.int32)
    wb = jnp.concatenate(
        [W.reshape(D), jnp.broadcast_to(b.reshape(1), (16,))]
    )
    parts = _sc_partials(feats_node, ids, wb)
    tcp = _tc_partial(
        feats_node, ids.reshape(1, N), W.reshape(1, D), b.reshape(1)
    )
    return _tc_combine(parts, tcp)


# speedup vs baseline: 1.1394x; 1.0012x over previous
import functools

import jax
import jax.numpy as jnp
from jax import lax
from jax.experimental import pallas as pl
from jax.experimental.pallas import tpu as pltpu
from jax.experimental.pallas import tpu_sc as plsc

from jax._src import core as _jax_core
from jax._src.lib.mlir import ir as _ir
from jax._src.lib.mlir.dialects import arith as _arith
from jax._src.lib.mlir.dialects import vector as _vector
from jax._src.pallas.mosaic import core as _tpu_core
from jax._src.pallas.mosaic import lowering as _tc_lowering
from jax.experimental.mosaic.dialects import tpu as _tpu_d

_lane_xor_p = _jax_core.Primitive("sc_lane_xor_perm")


@_lane_xor_p.def_abstract_eval
def _lane_xor_abstract_eval(x, *, k):
    del k
    return x


@_tc_lowering.register_lowering_rule(
    _lane_xor_p, kernel_types=[_tpu_core.CoreType.SC_VECTOR_SUBCORE]
)
def _lane_xor_lowering(ctx, x, *, k):
    del ctx
    i32 = _ir.IntegerType.get_signless(32)
    vec_ty = _ir.VectorType.get([16], i32)
    c = _arith.constant(i32, _ir.IntegerAttr.get(i32, k))
    cv = _vector.broadcast(vec_ty, c)
    idx = _arith.xori(cv, _tpu_d.iota(vec_ty, dimensions=[0]))
    return _tpu_d.dynamic_gather(x, idx, dimensions=[0])


def _lane_xor(x, k):
    return _lane_xor_p.bind(x, k=k)

N = 100000
D = 128
S = 512
RB = 128
NW = 32
BPW = 13
M = BPW * NW * RB
NT = N - M
BT = 4096
GT = (NT + BT - 1) // BT
WIDTH = D + 16
WT = D + 8

_mesh = plsc.VectorSubcoreMesh(
    core_axis_name="c", subcore_axis_name="s", num_cores=2, num_subcores=16
)


@functools.partial(
    pl.kernel,
    out_type=jax.ShapeDtypeStruct((2, S, WIDTH), jnp.float32),
    mesh=_mesh,
    scratch_types=[
        pltpu.VMEM((2, RB, D), jnp.float32),
        pltpu.VMEM((2, RB), jnp.int32),
        pltpu.VMEM((2, RB, WIDTH), jnp.float32),
        pltpu.VMEM((WIDTH,), jnp.float32),
        pltpu.VMEM_SHARED((S, WIDTH), jnp.float32),
        pltpu.SemaphoreType.DMA((2,)),
        pltpu.SemaphoreType.DMA((2,)),
        pltpu.SemaphoreType.DMA((2,)),
    ],
    compiler_params=pltpu.CompilerParams(use_tc_tiling_on_sc=False),
)
def _sc_partials(
    feats_hbm, ids_hbm, wb_hbm, out_hbm,
    fbuf, ibuf, sbuf, wbuf, acc, dfsem, disem, ssem,
):
    c = lax.axis_index("c")
    s = lax.axis_index("s")
    w = c * 16 + s

    for r in range(32):
        for j in range(WIDTH // 16):
            sbuf[0, r, pl.ds(16 * j, 16)] = jnp.zeros((16,), jnp.float32)
    pltpu.sync_copy(sbuf.at[0, pl.ds(0, 32)], acc.at[pl.ds(s * 32, 32)])
    plsc.subcore_barrier()

    pltpu.sync_copy(wb_hbm, wbuf)
    wv = [wbuf[pl.ds(16 * j, 16)] for j in range(8)]
    bv = wbuf[pl.ds(128, 16)]
    dmask = lax.iota(jnp.int32, 16) == 0

    def lane_allsum(a):
        for k in (1, 2, 4, 8):
            a = a + _lane_xor(a, k)
        return a

    nblk = BPW

    def issue_in(t, buf):
        row0 = (w + 32 * t) * RB
        pltpu.async_copy(
            feats_hbm.at[pl.ds(row0, RB)], fbuf.at[buf], dfsem.at[buf]
        )
        pltpu.async_copy(ids_hbm.at[pl.ds(row0, RB)], ibuf.at[buf], disem.at[buf])

    issue_in(0, 0)

    def blk_body(t, carry):
        buf = lax.rem(t, 2)
        nbuf = 1 - buf

        @pl.when(t >= 1)
        def _():
            pltpu.make_async_copy(
                sbuf.at[nbuf], acc.at[ibuf.at[nbuf]], ssem.at[nbuf]
            ).wait()

        @pl.when(t + 1 < nblk)
        def _():
            issue_in(t + 1, nbuf)

        pltpu.make_async_copy(
            feats_hbm.at[pl.ds(0, RB)], fbuf.at[buf], dfsem.at[buf]
        ).wait()
        pltpu.make_async_copy(
            ids_hbm.at[pl.ds(0, RB)], ibuf.at[buf], disem.at[buf]
        ).wait()

        def group_body(g, carry2):
            rows = []
            for k in range(8):
                r = g * 8 + k
                fs = [fbuf[buf, r, pl.ds(16 * j, 16)] for j in range(8)]
                p0 = fs[0] * wv[0]
                p1 = fs[1] * wv[1]
                for j in range(2, 8, 2):
                    p0 = p0 + fs[j] * wv[j]
                    p1 = p1 + fs[j + 1] * wv[j + 1]
                a = p0 + p1
                rows.append((r, fs, a))
            for r, fs, a in rows:
                sv = lane_allsum(a) + bv
                w16 = 1.0 / (1.0 + jnp.exp(-sv))
                for j in range(8):
                    sbuf[buf, r, pl.ds(16 * j, 16)] = fs[j] * w16
                sbuf[buf, r, pl.ds(128, 16)] = jnp.where(dmask, w16, 0.0)
            return carry2

        lax.fori_loop(0, RB // 8, group_body, 0)
        pltpu.async_copy(sbuf.at[buf], acc.at[ibuf.at[buf]], ssem.at[buf], add=True)
        return carry

    lax.fori_loop(0, nblk, blk_body, 0)
    lastbuf = lax.rem(nblk - 1, 2)
    pltpu.make_async_copy(
        sbuf.at[lastbuf], acc.at[ibuf.at[lastbuf]], ssem.at[lastbuf]
    ).wait()
    plsc.subcore_barrier()
    pltpu.sync_copy(acc.at[pl.ds(s * 32, 32)], out_hbm.at[c, pl.ds(s * 32, 32)])


def _tc_partial_body(feats_ref, ids_ref, w_ref, b_ref, out_ref):
    i = pl.program_id(0)
    f = feats_ref[...]
    logits = lax.dot_general(
        w_ref[...], f, (((1,), (1,)), ((), ())),
        preferred_element_type=jnp.float32,
    )
    wt = 1.0 / (1.0 + jnp.exp(-(logits + b_ref[0])))
    valid = (lax.broadcasted_iota(jnp.int32, (1, BT), 1) + i * BT) < NT
    wt = jnp.where(valid, wt, 0.0)
    wtc = wt.reshape(BT, 1)
    gid = lax.broadcasted_iota(jnp.int32, (S, BT), 0)
    onehot = jnp.where(gid == ids_ref[...], 1.0, 0.0)
    aug = jnp.concatenate(
        [f * wtc, jnp.broadcast_to(wtc, (BT, 8))], axis=1
    )
    nd = jnp.dot(onehot, aug, preferred_element_type=jnp.float32)

    @pl.when(i == 0)
    def _():
        out_ref[...] = jnp.zeros((S, WT), jnp.float32)

    out_ref[...] += nd


_tc_partial = pl.pallas_call(
    _tc_partial_body,
    grid=(GT,),
    in_specs=[
        pl.BlockSpec((BT, D), lambda i: (M // BT + i, 0)),
        pl.BlockSpec((1, BT), lambda i: (0, M // BT + i)),
        pl.BlockSpec((1, D), lambda i: (0, 0)),
        pl.BlockSpec(memory_space=pltpu.SMEM),
    ],
    out_specs=pl.BlockSpec((S, WT), lambda i: (0, 0)),
    out_shape=jax.ShapeDtypeStruct((S, WT), jnp.float32),
)


def _tc_combine_body(parts_ref, tcp_ref, out_ref):
    p = parts_ref[0] + parts_ref[1]
    t = tcp_ref[...]
    num = p[:, :D] + t[:, :D]
    den = p[:, D:D + 1] + t[:, D:D + 1]
    out_ref[...] = num / (den + 1e-12)


_tc_combine = pl.pallas_call(
    _tc_combine_body,
    in_specs=[
        pl.BlockSpec(memory_space=pltpu.MemorySpace.VMEM),
        pl.BlockSpec(memory_space=pltpu.MemorySpace.VMEM),
    ],
    out_shape=jax.ShapeDtypeStruct((S, D), jnp.float32),
)


@jax.jit
def kernel(feats_node, segment_ids, W, b):
    ids = segment_ids.astype(jnp.int32)
    wb = jnp.concatenate(
        [W.reshape(D), jnp.broadcast_to(b.reshape(1), (16,))]
    )
    parts = _sc_partials(feats_node, ids, wb)
    tcp = _tc_partial(
        feats_node, ids.reshape(1, N), W.reshape(1, D), b.reshape(1)
    )
    return _tc_combine(parts, tcp)
